# baseline (speedup 1.0000x reference)
import jax
import jax.numpy as jnp
from jax.experimental import pallas as pl
from jax.experimental.pallas import tpu as pltpu

N = 50000
DIM = 512
G = 64
BLK = 2000
NBLK = N // BLK
SUBBLK = 400
SUB = BLK // SUBBLK
NEG_INF = float("-inf")


def _fused_kernel(lims_ref, bcol_ref, x_ref, W1_ref, b1_ref, W2_ref, b2_ref,
                  W3_ref, b3_ref, out_ref,
                  sums_ref, cnts_ref, maxs_ref, gate_ref):
    p = pl.program_id(0)
    j = pl.program_id(1)

    @pl.when((p == 0) & (j == 0))
    def _init():
        sums_ref[...] = jnp.zeros_like(sums_ref)
        cnts_ref[...] = jnp.zeros_like(cnts_ref)
        maxs_ref[...] = jnp.full_like(maxs_ref, NEG_INF)

    bcol = bcol_ref[...]
    oh = (bcol == jax.lax.broadcasted_iota(jnp.int32, (BLK, G), 1)
          ).astype(jnp.float32)

    @pl.when(p == 0)
    def _phase0():
        x = x_ref[...]
        h = jnp.dot(x, W1_ref[...], preferred_element_type=jnp.float32) + b1_ref[...]
        h = jnp.where(h >= 0.0, h, 0.2 * h)
        latent = jnp.dot(h, W2_ref[...], preferred_element_type=jnp.float32) + b2_ref[...]

        sums_ref[...] += jax.lax.dot_general(
            oh, latent, (((0,), (0,)), ((), ())),
            preferred_element_type=jnp.float32)
        cnt = jnp.sum(oh, axis=0, keepdims=True)
        cnts_ref[...] += jnp.broadcast_to(cnt.T, cnts_ref.shape)

        for c in range(SUB):
            sub_lat = latent[c * SUBBLK:(c + 1) * SUBBLK, :]
            sub_b = bcol[c * SUBBLK:(c + 1) * SUBBLK, :]
            first = lims_ref[0, 0, 2 * c]
            last = lims_ref[0, 0, 2 * c + 1]

            def body(g, _, sub_b=sub_b, sub_lat=sub_lat):
                m = jnp.max(jnp.where(sub_b == g, sub_lat, NEG_INF),
                            axis=0, keepdims=True)
                maxs_ref[pl.ds(g, 1), :] = jnp.maximum(maxs_ref[pl.ds(g, 1), :], m)
                return 0

            jax.lax.fori_loop(first, last + 1, body, 0)

    @pl.when((p == 1) & (j == 0))
    def _gate():
        cnt = cnts_ref[:, :1]
        mean = sums_ref[...] / jnp.maximum(cnt, 1.0)
        mx = jnp.where(cnt > 0.0, maxs_ref[...], 0.0)
        z = (jnp.dot(mean, W3_ref[:DIM, :], preferred_element_type=jnp.float32)
             + jnp.dot(mx, W3_ref[DIM:, :], preferred_element_type=jnp.float32)
             + b3_ref[...])
        gate_ref[...] = jax.nn.sigmoid(z)

    @pl.when(p == 1)
    def _phase1():
        out_ref[...] = x_ref[...] * jnp.dot(oh, gate_ref[...],
                                            preferred_element_type=jnp.float32)


@jax.jit
def kernel(x, batch, W1, b1, W2, b2, W3, b3):
    b32 = batch.astype(jnp.int32)
    bcol = b32.reshape(N, 1)
    lims = jnp.stack([b32[0::SUBBLK], b32[SUBBLK - 1::SUBBLK]],
                     axis=1).reshape(NBLK, 1, 2 * SUB)
    b1r = b1.reshape(1, DIM)
    b2r = b2.reshape(1, DIM)
    b3r = b3.reshape(1, DIM)

    out = pl.pallas_call(
        _fused_kernel,
        grid=(2, NBLK),
        in_specs=[
            pl.BlockSpec((1, 1, 2 * SUB), lambda p, j: (j, 0, 0),
                         memory_space=pltpu.SMEM),
            pl.BlockSpec((BLK, 1), lambda p, j: (j, 0)),
            pl.BlockSpec((BLK, DIM), lambda p, j: (j, 0)),
            pl.BlockSpec((DIM, DIM), lambda p, j: (0, 0)),
            pl.BlockSpec((1, DIM), lambda p, j: (0, 0)),
            pl.BlockSpec((DIM, DIM), lambda p, j: (0, 0)),
            pl.BlockSpec((1, DIM), lambda p, j: (0, 0)),
            pl.BlockSpec((2 * DIM, DIM), lambda p, j: (0, 0)),
            pl.BlockSpec((1, DIM), lambda p, j: (0, 0)),
        ],
        out_specs=pl.BlockSpec((BLK, DIM), lambda p, j: (p * j, 0)),
        out_shape=jax.ShapeDtypeStruct((N, DIM), jnp.float32),
        scratch_shapes=[
            pltpu.VMEM((G, DIM), jnp.float32),
            pltpu.VMEM((G, 128), jnp.float32),
            pltpu.VMEM((G, DIM), jnp.float32),
            pltpu.VMEM((G, DIM), jnp.float32),
        ],
    )(lims, bcol, x, W1, b1r, W2, b2r, W3, b3r)

    return out


# speedup vs baseline: 1.0590x; 1.0590x over previous
import jax
import jax.numpy as jnp
from jax.experimental import pallas as pl
from jax.experimental.pallas import tpu as pltpu

N = 50000
DIM = 512
G = 64
BLK = 2000
NBLK = N // BLK
BLK2 = 5000
NBLK2 = N // BLK2
NEG_INF = float("-inf")
SUBBLK = 400
SUB = BLK // SUBBLK


def _pass1_kernel(lims_ref, bcol_ref, x_ref, W1_ref, b1_ref, W2_ref, b2_ref,
                  sums_ref, cnts_ref, maxs_ref):
    i = pl.program_id(0)

    @pl.when(i == 0)
    def _init():
        sums_ref[...] = jnp.zeros_like(sums_ref)
        cnts_ref[...] = jnp.zeros_like(cnts_ref)
        maxs_ref[...] = jnp.full_like(maxs_ref, NEG_INF)

    x = x_ref[...]
    h = jnp.dot(x, W1_ref[...], preferred_element_type=jnp.float32) + b1_ref[...]
    h = jnp.where(h >= 0.0, h, 0.2 * h)
    latent = jnp.dot(h, W2_ref[...], preferred_element_type=jnp.float32) + b2_ref[...]

    bcol = bcol_ref[...]
    oh = (bcol == jax.lax.broadcasted_iota(jnp.int32, (BLK, G), 1)
          ).astype(jnp.float32)
    sums_ref[...] += jax.lax.dot_general(
        oh, latent, (((0,), (0,)), ((), ())),
        preferred_element_type=jnp.float32)
    cnt = jnp.sum(oh, axis=0, keepdims=True)
    cnts_ref[...] += jnp.broadcast_to(cnt.T, cnts_ref.shape)

    for c in range(SUB):
        sub_lat = latent[c * SUBBLK:(c + 1) * SUBBLK, :]
        sub_b = bcol[c * SUBBLK:(c + 1) * SUBBLK, :]
        first = lims_ref[0, 0, 2 * c]
        last = lims_ref[0, 0, 2 * c + 1]

        def body(g, _, sub_b=sub_b, sub_lat=sub_lat):
            m = jnp.max(jnp.where(sub_b == g, sub_lat, NEG_INF),
                        axis=0, keepdims=True)
            maxs_ref[pl.ds(g, 1), :] = jnp.maximum(maxs_ref[pl.ds(g, 1), :], m)
            return 0

        jax.lax.fori_loop(first, last + 1, body, 0)


def _pass2_kernel(bcol_ref, x_ref, sums_ref, cnts_ref, maxs_ref,
                  W3_ref, b3_ref, out_ref, gate_ref):
    i = pl.program_id(0)

    @pl.when(i == 0)
    def _gate():
        cnt = cnts_ref[:, :1]
        mean = sums_ref[...] / jnp.maximum(cnt, 1.0)
        mx = jnp.where(cnt > 0.0, maxs_ref[...], 0.0)
        z = (jnp.dot(mean, W3_ref[:DIM, :], preferred_element_type=jnp.float32)
             + jnp.dot(mx, W3_ref[DIM:, :], preferred_element_type=jnp.float32)
             + b3_ref[...])
        gate_ref[...] = jax.nn.sigmoid(z)

    bcol = bcol_ref[...]
    oh = (bcol == jax.lax.broadcasted_iota(jnp.int32, (BLK2, G), 1)
          ).astype(jnp.float32)
    out_ref[...] = x_ref[...] * jnp.dot(oh, gate_ref[...],
                                        preferred_element_type=jnp.float32)


@jax.jit
def kernel(x, batch, W1, b1, W2, b2, W3, b3):
    b32 = batch.astype(jnp.int32)
    bcol = b32.reshape(N, 1)
    lims = jnp.stack([b32[0::SUBBLK], b32[SUBBLK - 1::SUBBLK]],
                     axis=1).reshape(NBLK, 1, 2 * SUB)
    b1r = b1.reshape(1, DIM)
    b2r = b2.reshape(1, DIM)
    b3r = b3.reshape(1, DIM)

    sums, cnts, maxs = pl.pallas_call(
        _pass1_kernel,
        grid=(NBLK,),
        in_specs=[
            pl.BlockSpec((1, 1, 2 * SUB), lambda i: (i, 0, 0),
                         memory_space=pltpu.SMEM),
            pl.BlockSpec((BLK, 1), lambda i: (i, 0)),
            pl.BlockSpec((BLK, DIM), lambda i: (i, 0)),
            pl.BlockSpec((DIM, DIM), lambda i: (0, 0)),
            pl.BlockSpec((1, DIM), lambda i: (0, 0)),
            pl.BlockSpec((DIM, DIM), lambda i: (0, 0)),
            pl.BlockSpec((1, DIM), lambda i: (0, 0)),
        ],
        out_specs=[
            pl.BlockSpec((G, DIM), lambda i: (0, 0)),
            pl.BlockSpec((G, 128), lambda i: (0, 0)),
            pl.BlockSpec((G, DIM), lambda i: (0, 0)),
        ],
        out_shape=[
            jax.ShapeDtypeStruct((G, DIM), jnp.float32),
            jax.ShapeDtypeStruct((G, 128), jnp.float32),
            jax.ShapeDtypeStruct((G, DIM), jnp.float32),
        ],
    )(lims, bcol, x, W1, b1r, W2, b2r)

    out = pl.pallas_call(
        _pass2_kernel,
        grid=(NBLK2,),
        in_specs=[
            pl.BlockSpec((BLK2, 1), lambda i: (i, 0)),
            pl.BlockSpec((BLK2, DIM), lambda i: (i, 0)),
            pl.BlockSpec((G, DIM), lambda i: (0, 0)),
            pl.BlockSpec((G, 128), lambda i: (0, 0)),
            pl.BlockSpec((G, DIM), lambda i: (0, 0)),
            pl.BlockSpec((2 * DIM, DIM), lambda i: (0, 0)),
            pl.BlockSpec((1, DIM), lambda i: (0, 0)),
        ],
        out_specs=pl.BlockSpec((BLK2, DIM), lambda i: (i, 0)),
        out_shape=jax.ShapeDtypeStruct((N, DIM), jnp.float32),
        scratch_shapes=[pltpu.VMEM((G, DIM), jnp.float32)],
    )(bcol, x, sums, cnts, maxs, W3, b3r)

    return out
